# baseline (speedup 1.0000x reference)
import functools

import jax
import jax.numpy as jnp
from jax import lax
from jax.experimental import pallas as pl
from jax.experimental.pallas import tpu as pltpu
from jax.experimental.pallas import tpu_sc as plsc

_NUM_WORKERS = 32
_SEQ_PAD = 56


def _mm_body(e_ref, w_ref, b_ref, out_ref):
    out_ref[...] = lax.dot_general(
        e_ref[...], w_ref[...],
        dimension_numbers=(((1,), (1,)), ((), ())),
        preferred_element_type=jnp.float32,
    ) + b_ref[...]


def _fused_table(embedding, W, b):
    vocab = W.shape[0]
    return pl.pallas_call(
        _mm_body,
        out_shape=jax.ShapeDtypeStruct((embedding.shape[0], vocab), jnp.float32),
    )(embedding, W, b.reshape(1, vocab))


@functools.lru_cache(maxsize=None)
def _make_gather(bsz, seq, vocab):
    rows_per_w = bsz // _NUM_WORKERS
    mesh = plsc.VectorSubcoreMesh(core_axis_name="c", subcore_axis_name="s")

    @functools.partial(
        pl.kernel,
        mesh=mesh,
        out_type=jax.ShapeDtypeStruct((bsz, seq, vocab), jnp.float32),
        scratch_types=[
            pltpu.VMEM((rows_per_w * _SEQ_PAD,), jnp.int32),
            pltpu.VMEM((seq, vocab), jnp.float32),
            pltpu.SemaphoreType.DMA,
        ],
        compiler_params=pltpu.CompilerParams(use_tc_tiling_on_sc=False),
    )
    def gather_k(table_hbm, idx_hbm, out_hbm, idx_v, rows_v, sem):
        wid = lax.axis_index("s") * 2 + lax.axis_index("c")
        row0 = wid * rows_per_w
        pltpu.sync_copy(
            idx_hbm.at[pl.ds(row0 * _SEQ_PAD, rows_per_w * _SEQ_PAD)], idx_v
        )

        def body(i, carry):
            pltpu.async_copy(
                table_hbm.at[idx_v.at[pl.ds(i * _SEQ_PAD, seq)]], rows_v, sem
            ).wait()
            pltpu.sync_copy(rows_v, out_hbm.at[row0 + i])
            return carry

        lax.fori_loop(0, rows_per_w, body, 0)

    return gather_k


def kernel(input_ids, embedding, W, b):
    bsz, seq = input_ids.shape
    vocab = W.shape[0]
    ids = jnp.pad(input_ids.astype(jnp.int32), ((0, 0), (0, _SEQ_PAD - seq)))
    ids = ids.reshape(-1)
    table = _fused_table(embedding, W, b)
    return _make_gather(bsz, seq, vocab)(table, ids)


# speedup vs baseline: 1.0107x; 1.0107x over previous
import functools

import jax
import jax.numpy as jnp
from jax import lax
from jax.experimental import pallas as pl
from jax.experimental.pallas import tpu as pltpu
from jax.experimental.pallas import tpu_sc as plsc

_NUM_WORKERS = 32
_CHUNK = 40


def _mm_body(e_ref, w_ref, b_ref, out_ref):
    out_ref[...] = lax.dot_general(
        e_ref[...], w_ref[...],
        dimension_numbers=(((1,), (1,)), ((), ())),
        preferred_element_type=jnp.float32,
    ) + b_ref[...]


def _fused_table(embedding, W, b):
    vocab = W.shape[0]
    return pl.pallas_call(
        _mm_body,
        out_shape=jax.ShapeDtypeStruct((embedding.shape[0], vocab), jnp.float32),
    )(embedding, W, b.reshape(1, vocab))


@functools.lru_cache(maxsize=None)
def _make_gather(n_tokens, vocab):
    toks_per_w = n_tokens // _NUM_WORKERS
    n_chunks = toks_per_w // _CHUNK
    mesh = plsc.VectorSubcoreMesh(core_axis_name="c", subcore_axis_name="s")

    @functools.partial(
        pl.kernel,
        mesh=mesh,
        out_type=jax.ShapeDtypeStruct((n_tokens, vocab), jnp.float32),
        scratch_types=[
            pltpu.VMEM((toks_per_w,), jnp.int32),
            pltpu.VMEM((_CHUNK, vocab), jnp.float32),
            pltpu.VMEM((_CHUNK, vocab), jnp.float32),
            pltpu.SemaphoreType.DMA,
            pltpu.SemaphoreType.DMA,
            pltpu.SemaphoreType.DMA,
            pltpu.SemaphoreType.DMA,
        ],
        compiler_params=pltpu.CompilerParams(use_tc_tiling_on_sc=False),
    )
    def gather_k(table_hbm, idx_hbm, out_hbm,
                 idx_v, buf0, buf1, gsem0, gsem1, psem0, psem1):
        wid = lax.axis_index("s") * 2 + lax.axis_index("c")
        tok0 = wid * toks_per_w
        pltpu.sync_copy(idx_hbm.at[pl.ds(tok0, toks_per_w)], idx_v)

        def gather(i, buf, sem):
            return pltpu.async_copy(
                table_hbm.at[idx_v.at[pl.ds(i * _CHUNK, _CHUNK)]], buf, sem)

        def put(i, buf, sem):
            return pltpu.async_copy(
                buf, out_hbm.at[pl.ds(tok0 + i * _CHUNK, _CHUNK)], sem)

        gather(0, buf0, gsem0)
        gather(1, buf1, gsem1)

        def pair(j, carry):
            i = 2 * j
            pltpu.make_async_copy(
                table_hbm.at[idx_v.at[pl.ds(0, _CHUNK)]], buf0, gsem0
            ).wait()
            put(i, buf0, psem0)
            pltpu.make_async_copy(
                table_hbm.at[idx_v.at[pl.ds(0, _CHUNK)]], buf1, gsem1
            ).wait()
            put(i + 1, buf1, psem1)
            pltpu.make_async_copy(
                buf0, out_hbm.at[pl.ds(tok0, _CHUNK)], psem0
            ).wait()
            gather(i + 2, buf0, gsem0)
            pltpu.make_async_copy(
                buf1, out_hbm.at[pl.ds(tok0, _CHUNK)], psem1
            ).wait()
            gather(i + 3, buf1, gsem1)
            return carry

        lax.fori_loop(0, n_chunks // 2 - 1, pair, 0)

        i = n_chunks - 2
        pltpu.make_async_copy(
            table_hbm.at[idx_v.at[pl.ds(0, _CHUNK)]], buf0, gsem0).wait()
        put(i, buf0, psem0)
        pltpu.make_async_copy(
            table_hbm.at[idx_v.at[pl.ds(0, _CHUNK)]], buf1, gsem1).wait()
        put(i + 1, buf1, psem1)
        pltpu.make_async_copy(
            buf0, out_hbm.at[pl.ds(tok0, _CHUNK)], psem0).wait()
        pltpu.make_async_copy(
            buf1, out_hbm.at[pl.ds(tok0, _CHUNK)], psem1).wait()

    return gather_k


def kernel(input_ids, embedding, W, b):
    bsz, seq = input_ids.shape
    vocab = W.shape[0]
    ids = input_ids.astype(jnp.int32).reshape(-1)
    table = _fused_table(embedding, W, b)
    flat = _make_gather(bsz * seq, vocab)(table, ids)
    return flat.reshape(bsz, seq, vocab)
